# Initial kernel scaffold; baseline (speedup 1.0000x reference)
import jax
import jax.numpy as jnp
from jax import lax
from jax.experimental import pallas as pl
from jax.experimental.pallas import tpu as pltpu
from jax.experimental.pallas import tpu_sc as plsc

NUM_EMBEDDINGS = 1000000
EMBEDDING_DIM = 64
BATCH = 16384
SEQ_LEN = 50

_INFO = plsc.get_sparse_core_info()
NC, NS = _INFO.num_cores, _INFO.num_subcores
NW = NC * NS

N_ROWS = BATCH * SEQ_LEN
PER_W = N_ROWS // NW
IDX_MINOR = 128
N_IDX_ROWS = PER_W // IDX_MINOR
TILE = 512
G = TILE // IDX_MINOR
STEPS = PER_W // TILE
NBUF = 2


def _body(idx_hbm, table_hbm, out_hbm, idx_v, rows_v, sg0, sg1, ss0, ss1):
    wid = lax.axis_index("s") * NC + lax.axis_index("c")
    base = wid * PER_W
    sem_g = (sg0, sg1)
    sem_s = (ss0, ss1)

    pltpu.sync_copy(idx_hbm.at[wid], idx_v)

    def issue_gathers(s, b):
        for j in range(G):
            pltpu.async_copy(
                table_hbm.at[idx_v.at[s * G + j]],
                rows_v.at[b, pl.ds(j * IDX_MINOR, IDX_MINOR)],
                sem_g[b],
            )

    def wait_gathers(b):
        pltpu.make_async_copy(
            table_hbm.at[pl.ds(0, TILE)], rows_v.at[b], sem_g[b]
        ).wait()

    def wait_store(b):
        pltpu.make_async_copy(
            rows_v.at[b], out_hbm.at[pl.ds(base, TILE)], sem_s[b]
        ).wait()

    for b in range(NBUF):
        issue_gathers(b, b)

    def outer(g, carry):
        for b in range(NBUF):
            s = g * NBUF + b
            wait_gathers(b)
            pltpu.async_copy(
                rows_v.at[b], out_hbm.at[pl.ds(base + s * TILE, TILE)], sem_s[b]
            )
            wait_store(b)

            @pl.when(s + NBUF < STEPS)
            def _():
                issue_gathers(s + NBUF, b)

        return carry

    lax.fori_loop(0, STEPS // NBUF, outer, 0)


@jax.jit
def _gather(token_ids_flat, table):
    mesh = plsc.VectorSubcoreMesh(core_axis_name="c", subcore_axis_name="s")
    run = pl.kernel(
        _body,
        out_type=jax.ShapeDtypeStruct((N_ROWS, EMBEDDING_DIM), jnp.float32),
        mesh=mesh,
        scratch_types=[
            pltpu.VMEM((N_IDX_ROWS, IDX_MINOR), jnp.int32),
            pltpu.VMEM((NBUF, TILE, EMBEDDING_DIM), jnp.float32),
            pltpu.SemaphoreType.DMA,
            pltpu.SemaphoreType.DMA,
            pltpu.SemaphoreType.DMA,
            pltpu.SemaphoreType.DMA,
        ],
    )
    return run(token_ids_flat, table)


def kernel(token_ids, table):
    idx = token_ids.astype(jnp.int32).reshape(NW, N_IDX_ROWS, IDX_MINOR)
    out = _gather(idx, table)
    return out.reshape(BATCH, SEQ_LEN, EMBEDDING_DIM)


# speedup vs baseline: 1.8754x; 1.8754x over previous
import jax
import jax.numpy as jnp
from jax import lax
from jax.experimental import pallas as pl
from jax.experimental.pallas import tpu as pltpu
from jax.experimental.pallas import tpu_sc as plsc

NUM_EMBEDDINGS = 1000000
EMBEDDING_DIM = 64
BATCH = 16384
SEQ_LEN = 50

_INFO = plsc.get_sparse_core_info()
NC, NS = _INFO.num_cores, _INFO.num_subcores
NW = NC * NS

N_ROWS = BATCH * SEQ_LEN
PER_W = N_ROWS // NW
IDX_MINOR = 128
N_IDX_ROWS = PER_W // IDX_MINOR
TILE = 512
G = TILE // IDX_MINOR
STEPS = PER_W // TILE
NBUF = 2


def _body(idx_hbm, table_hbm, out_hbm, idx_v, rows_v, sg0, sg1, ss0, ss1):
    wid = lax.axis_index("s") * NC + lax.axis_index("c")
    base = wid * PER_W
    sem_g = (sg0, sg1)
    sem_s = (ss0, ss1)

    pltpu.sync_copy(idx_hbm.at[wid], idx_v)

    def issue_gathers(s, b):
        for j in range(G):
            pltpu.async_copy(
                table_hbm.at[idx_v.at[s * G + j]],
                rows_v.at[b, pl.ds(j * IDX_MINOR, IDX_MINOR)],
                sem_g[b],
            )

    def wait_gathers(b):
        pltpu.make_async_copy(
            table_hbm.at[pl.ds(0, TILE)], rows_v.at[b], sem_g[b]
        ).wait()

    def wait_store(b):
        pltpu.make_async_copy(
            rows_v.at[b], out_hbm.at[pl.ds(base, TILE)], sem_s[b]
        ).wait()

    for b in range(NBUF):
        issue_gathers(b, b)

    def outer(g, carry):
        for b in range(NBUF):
            s = g * NBUF + b
            wait_gathers(b)
            pltpu.async_copy(
                rows_v.at[b], out_hbm.at[pl.ds(base + s * TILE, TILE)], sem_s[b]
            )
            wait_store(b)

            @pl.when(s + NBUF < STEPS)
            def _():
                issue_gathers(s + NBUF, b)

        return carry

    lax.fori_loop(0, STEPS // NBUF, outer, 0)


@jax.jit
def _gather(token_ids_flat, table):
    mesh = plsc.VectorSubcoreMesh(core_axis_name="c", subcore_axis_name="s")
    run = pl.kernel(
        _body,
        out_type=jax.ShapeDtypeStruct((N_ROWS, EMBEDDING_DIM), jnp.float32),
        mesh=mesh,
        scratch_types=[
            pltpu.VMEM((N_IDX_ROWS, IDX_MINOR), jnp.int32),
            pltpu.VMEM((NBUF, TILE, EMBEDDING_DIM), jnp.float32),
            pltpu.SemaphoreType.DMA,
            pltpu.SemaphoreType.DMA,
            pltpu.SemaphoreType.DMA,
            pltpu.SemaphoreType.DMA,
        ],
        compiler_params=pltpu.CompilerParams(use_tc_tiling_on_sc=False),
    )
    return run(token_ids_flat, table)


def kernel(token_ids, table):
    idx = token_ids.astype(jnp.int32).reshape(NW, N_IDX_ROWS, IDX_MINOR)
    out = _gather(idx, table)
    return out.reshape(BATCH, SEQ_LEN, EMBEDDING_DIM)
